# baseline (speedup 1.0000x reference)
import dataclasses
import functools

import jax
import jax.numpy as jnp
from jax import lax
from jax.experimental import pallas as pl
from jax.experimental.pallas import tpu as pltpu
from jax.experimental.pallas import tpu_sc as plsc

NC = 2
NS = 16
NW = NC * NS
LANES = 16
CHUNK = 128


def kernel(user_ids, item_ids, user_factors, item_factors, user_biases,
           item_biases, global_bias):
    B = user_ids.shape[0]
    D = user_factors.shape[1]
    bpw = B // NW
    nchunks = bpw // CHUNK

    uids = user_ids.astype(jnp.int32)
    iids = item_ids.astype(jnp.int32)
    upair = (uids >> 1).reshape(B // CHUNK, CHUNK)
    ipair = (iids >> 1).reshape(B // CHUNK, CHUNK)
    uids = uids.reshape(B // CHUNK, CHUNK)
    iids = iids.reshape(B // CHUNK, CHUNK)
    uf2 = user_factors.reshape(user_factors.shape[0] // 2, 2 * D)
    if2 = item_factors.reshape(item_factors.shape[0] // 2, 2 * D)
    ub_flat = user_biases.reshape(-1)
    ib_flat = item_biases.reshape(-1)
    gb_vec = jnp.broadcast_to(global_bias.reshape(()), (LANES,))

    mesh = plsc.VectorSubcoreMesh(core_axis_name="c", subcore_axis_name="s")

    cp = pltpu.CompilerParams()
    for field, val in (("needs_layout_passes", False),
                       ("use_tc_tiling_on_sc", True)):
        if field in pltpu.CompilerParams.__dataclass_fields__:
            cp = dataclasses.replace(cp, **{field: val})

    @functools.partial(
        pl.kernel,
        out_type=jax.ShapeDtypeStruct((B,), jnp.float32),
        mesh=mesh,
        compiler_params=cp,
        scratch_types=[
            pltpu.VMEM((nchunks, CHUNK), jnp.int32),
            pltpu.VMEM((nchunks, CHUNK), jnp.int32),
            pltpu.VMEM((nchunks, CHUNK), jnp.int32),
            pltpu.VMEM((nchunks, CHUNK), jnp.int32),
            pltpu.VMEM((2, CHUNK, 2 * D), jnp.float32),
            pltpu.VMEM((2, CHUNK, 2 * D), jnp.float32),
            pltpu.VMEM((bpw,), jnp.float32),
            pltpu.VMEM((bpw,), jnp.float32),
            pltpu.VMEM((bpw,), jnp.float32),
            pltpu.VMEM((LANES,), jnp.float32),
            pltpu.SemaphoreType.DMA,
            pltpu.SemaphoreType.DMA,
        ],
    )
    def mf_kernel(uid_hbm, iid_hbm, up_hbm, ip_hbm, uf_hbm, if_hbm, ub_hbm,
                  ib_hbm, gb_hbm, out_hbm, uid_v, iid_v, up_v, ip_v, urows,
                  irows, ubv, ibv, outv, gbv, sem_b, sem_f):
        wid = lax.axis_index("s") * NC + lax.axis_index("c")
        base = wid * bpw

        idx_rows = pl.ds(wid * nchunks, nchunks)
        pltpu.sync_copy(uid_hbm.at[idx_rows], uid_v)
        pltpu.sync_copy(iid_hbm.at[idx_rows], iid_v)
        pltpu.sync_copy(up_hbm.at[idx_rows], up_v)
        pltpu.sync_copy(ip_hbm.at[idx_rows], ip_v)
        pltpu.sync_copy(gb_hbm, gbv)

        bias_copies = []
        for j in range(nchunks):
            sl = pl.ds(j * CHUNK, CHUNK)
            bias_copies.append(
                pltpu.async_copy(ub_hbm.at[uid_v.at[j]], ubv.at[sl], sem_b))
            bias_copies.append(
                pltpu.async_copy(ib_hbm.at[iid_v.at[j]], ibv.at[sl], sem_b))

        def start_chunk(j):
            slot = j % 2
            return (pltpu.async_copy(uf_hbm.at[up_v.at[j]], urows.at[slot],
                                     sem_f),
                    pltpu.async_copy(if_hbm.at[ip_v.at[j]], irows.at[slot],
                                     sem_f))

        inflight = start_chunk(0)
        for cp_ in bias_copies:
            cp_.wait()
        gb = gbv[...]
        iota16 = lax.iota(jnp.int32, LANES)

        for j in range(nchunks):
            slot = j % 2
            for cp_ in inflight:
                cp_.wait()
            if j + 1 < nchunks:
                inflight = start_chunk(j + 1)
            u_slot = urows.at[slot]
            i_slot = irows.at[slot]

            @pl.loop(0, CHUNK // LANES)
            def _(g):
                csl = pl.ds(g * LANES, LANES)
                osl = pl.ds(j * CHUNK + g * LANES, LANES)
                row_idx = iota16 + g * LANES
                ucol = (uid_v[j, csl] & 1) << 6
                icol = (iid_v[j, csl] & 1) << 6
                acc = ubv[osl] + ibv[osl] + gb
                for d in range(D):
                    pu = plsc.load_gather(u_slot, [row_idx, ucol + d])
                    pv = plsc.load_gather(i_slot, [row_idx, icol + d])
                    acc = acc + pu * pv
                outv[osl] = acc

        pltpu.sync_copy(outv, out_hbm.at[pl.ds(base, bpw)])

    return mf_kernel(uids, iids, upair, ipair, uf2, if2, ub_flat, ib_flat,
                     gb_vec)


# speedup vs baseline: 1.4552x; 1.4552x over previous
import dataclasses
import functools

import jax
import jax.numpy as jnp
from jax import lax
from jax.experimental import pallas as pl
from jax.experimental.pallas import tpu as pltpu
from jax.experimental.pallas import tpu_sc as plsc

NC = 2
NS = 16
NW = NC * NS
LANES = 16
CHUNK = 128


def kernel(user_ids, item_ids, user_factors, item_factors, user_biases,
           item_biases, global_bias):
    B = user_ids.shape[0]
    D = user_factors.shape[1]
    bpw = B // NW
    nchunks = bpw // CHUNK
    gpc = CHUNK // LANES

    uids = user_ids.astype(jnp.int32).reshape(B // CHUNK, CHUNK)
    iids = item_ids.astype(jnp.int32).reshape(B // CHUNK, CHUNK)
    ub_flat = user_biases.reshape(-1)
    ib_flat = item_biases.reshape(-1)
    gb_vec = jnp.broadcast_to(global_bias.reshape(()), (LANES,))

    mesh = plsc.VectorSubcoreMesh(core_axis_name="c", subcore_axis_name="s")

    cp = pltpu.CompilerParams()
    for field, val in (("needs_layout_passes", False),
                       ("use_tc_tiling_on_sc", True)):
        if field in pltpu.CompilerParams.__dataclass_fields__:
            cp = dataclasses.replace(cp, **{field: val})

    @functools.partial(
        pl.kernel,
        out_type=jax.ShapeDtypeStruct((B,), jnp.float32),
        mesh=mesh,
        compiler_params=cp,
        scratch_types=[
            pltpu.VMEM((nchunks, CHUNK), jnp.int32),
            pltpu.VMEM((nchunks, CHUNK), jnp.int32),
            pltpu.VMEM((2, CHUNK, D), jnp.float32),
            pltpu.VMEM((2, CHUNK, D), jnp.float32),
            pltpu.VMEM((bpw,), jnp.float32),
            pltpu.VMEM((bpw,), jnp.float32),
            pltpu.VMEM((bpw,), jnp.float32),
            pltpu.VMEM((LANES,), jnp.float32),
            pltpu.SemaphoreType.DMA,
            pltpu.SemaphoreType.DMA,
        ],
    )
    def mf_kernel(uid_hbm, iid_hbm, uf_hbm, if_hbm, ub_hbm, ib_hbm, gb_hbm,
                  out_hbm, uid_v, iid_v, urows, irows, ubv, ibv, outv, gbv,
                  sem_b, sem_f):
        wid = lax.axis_index("s") * NC + lax.axis_index("c")
        base = wid * bpw

        idx_rows = pl.ds(wid * nchunks, nchunks)
        pltpu.sync_copy(uid_hbm.at[idx_rows], uid_v)
        pltpu.sync_copy(iid_hbm.at[idx_rows], iid_v)
        pltpu.sync_copy(gb_hbm, gbv)

        bias_copies = []
        for j in range(nchunks):
            sl = pl.ds(j * CHUNK, CHUNK)
            bias_copies.append(
                pltpu.async_copy(ub_hbm.at[uid_v.at[j]], ubv.at[sl], sem_b))
            bias_copies.append(
                pltpu.async_copy(ib_hbm.at[iid_v.at[j]], ibv.at[sl], sem_b))

        iota16 = lax.iota(jnp.int32, LANES)

        def issue_chunk(j, slot):
            @pl.loop(0, gpc)
            def _(g):
                uvec = uid_v[j, pl.ds(g * LANES, LANES)]
                ivec = iid_v[j, pl.ds(g * LANES, LANES)]
                for lane in range(LANES):
                    rid_u = jnp.sum(jnp.where(iota16 == lane, uvec, 0))
                    rid_i = jnp.sum(jnp.where(iota16 == lane, ivec, 0))
                    dst = pl.ds(g * LANES + lane, 1)
                    pltpu.async_copy(uf_hbm.at[pl.ds(rid_u, 1)],
                                     urows.at[slot].at[dst], sem_f)
                    pltpu.async_copy(if_hbm.at[pl.ds(rid_i, 1)],
                                     irows.at[slot].at[dst], sem_f)

        def drain_chunk(slot):
            pltpu.make_async_copy(uf_hbm.at[pl.ds(0, CHUNK)],
                                  urows.at[slot], sem_f).wait()
            pltpu.make_async_copy(if_hbm.at[pl.ds(0, CHUNK)],
                                  irows.at[slot], sem_f).wait()

        issue_chunk(0, 0)
        for cp_ in bias_copies:
            cp_.wait()
        gb = gbv[...]

        for j in range(nchunks):
            slot = j % 2
            drain_chunk(slot)
            if j + 1 < nchunks:
                issue_chunk(j + 1, (j + 1) % 2)
            u_slot = urows.at[slot]
            i_slot = irows.at[slot]

            @pl.loop(0, gpc)
            def _(g):
                csl = pl.ds(g * LANES, LANES)
                osl = pl.ds(j * CHUNK + g * LANES, LANES)
                row_idx = iota16 + g * LANES
                acc = ubv[osl] + ibv[osl] + gb
                for d in range(D):
                    col_idx = jnp.full((LANES,), d, jnp.int32)
                    pu = plsc.load_gather(u_slot, [row_idx, col_idx])
                    pv = plsc.load_gather(i_slot, [row_idx, col_idx])
                    acc = acc + pu * pv
                outv[osl] = acc

        pltpu.sync_copy(outv, out_hbm.at[pl.ds(base, bpw)])

    return mf_kernel(uids, iids, user_factors, item_factors, ub_flat,
                     ib_flat, gb_vec)


# speedup vs baseline: 2.1225x; 1.4585x over previous
import dataclasses
import functools

import jax
import jax.numpy as jnp
from jax import lax
from jax.experimental import pallas as pl
from jax.experimental.pallas import tpu as pltpu
from jax.experimental.pallas import tpu_sc as plsc

NC = 2
NS = 16
NW = NC * NS
LANES = 16
CHUNK = 128
RING = 4


def kernel(user_ids, item_ids, user_factors, item_factors, user_biases,
           item_biases, global_bias):
    B = user_ids.shape[0]
    D = user_factors.shape[1]
    bpw = B // NW
    nchunks = bpw // CHUNK
    gpc = CHUNK // LANES

    uids = user_ids.astype(jnp.int32).reshape(B // CHUNK, CHUNK)
    iids = item_ids.astype(jnp.int32).reshape(B // CHUNK, CHUNK)
    uf_t = user_factors.T
    ub_flat = user_biases.reshape(-1)
    ib_flat = item_biases.reshape(-1)
    gb_vec = jnp.broadcast_to(global_bias.reshape(()), (LANES,))

    mesh = plsc.VectorSubcoreMesh(core_axis_name="c", subcore_axis_name="s")

    cp = pltpu.CompilerParams()
    for field, val in (("needs_layout_passes", False),
                       ("use_tc_tiling_on_sc", True)):
        if field in pltpu.CompilerParams.__dataclass_fields__:
            cp = dataclasses.replace(cp, **{field: val})

    @functools.partial(
        pl.kernel,
        out_type=jax.ShapeDtypeStruct((B,), jnp.float32),
        mesh=mesh,
        compiler_params=cp,
        scratch_types=[
            pltpu.VMEM((nchunks, CHUNK), jnp.int32),
            pltpu.VMEM((nchunks, CHUNK), jnp.int32),
            pltpu.VMEM((RING, D, CHUNK), jnp.float32),
            pltpu.VMEM((2, CHUNK, D), jnp.float32),
            pltpu.VMEM((bpw,), jnp.float32),
            pltpu.VMEM((bpw,), jnp.float32),
            pltpu.VMEM((bpw,), jnp.float32),
            pltpu.VMEM((LANES,), jnp.float32),
            pltpu.SemaphoreType.DMA,
            pltpu.SemaphoreType.DMA,
            pltpu.SemaphoreType.DMA,
            pltpu.SemaphoreType.DMA,
            pltpu.SemaphoreType.DMA,
            pltpu.SemaphoreType.DMA,
            pltpu.SemaphoreType.DMA,
        ],
    )
    def mf_kernel(uid_hbm, iid_hbm, uf_hbm, if_hbm, ub_hbm, ib_hbm, gb_hbm,
                  out_hbm, uid_v, iid_v, ublk, irows, ubv, ibv, outv, gbv,
                  sem_b, sem_i0, sem_i1, su0, su1, su2, su3):
        wid = lax.axis_index("s") * NC + lax.axis_index("c")
        base = wid * bpw
        isems = (sem_i0, sem_i1)
        usems = (su0, su1, su2, su3)

        idx_rows = pl.ds(wid * nchunks, nchunks)
        pltpu.sync_copy(uid_hbm.at[idx_rows], uid_v)
        pltpu.sync_copy(iid_hbm.at[idx_rows], iid_v)
        pltpu.sync_copy(gb_hbm, gbv)

        bias_copies = []
        for j in range(nchunks):
            sl = pl.ds(j * CHUNK, CHUNK)
            bias_copies.append(
                pltpu.async_copy(ub_hbm.at[uid_v.at[j]], ubv.at[sl], sem_b))
            bias_copies.append(
                pltpu.async_copy(ib_hbm.at[iid_v.at[j]], ibv.at[sl], sem_b))

        iota16 = lax.iota(jnp.int32, LANES)
        zeros16 = jnp.zeros((LANES,), jnp.int32)

        def uid_at(r):
            grp = (r >> 4) << 4
            lane = r & 15
            vec = uid_v[r >> 7, pl.ds(grp & 127, LANES)]
            return jnp.sum(jnp.where(iota16 == lane, vec, 0))

        def fire_user(r, s):
            rid = uid_at(r)
            blk = pl.multiple_of((rid >> 7) << 7, CHUNK)
            pltpu.async_copy(uf_hbm.at[:, pl.ds(blk, CHUNK)],
                             ublk.at[s], usems[s])

        def drain_user(s):
            pltpu.make_async_copy(uf_hbm.at[:, pl.ds(0, CHUNK)],
                                  ublk.at[s], usems[s]).wait()

        def fire_items(j, slot):
            @pl.loop(0, gpc)
            def _(g):
                ivec = iid_v[j, pl.ds(g * LANES, LANES)]
                for lane in range(LANES):
                    rid = jnp.sum(jnp.where(iota16 == lane, ivec, 0))
                    dst = pl.ds(g * LANES + lane, 1)
                    pltpu.async_copy(if_hbm.at[pl.ds(rid, 1)],
                                     irows.at[slot].at[dst], isems[slot])

        def drain_items(slot):
            pltpu.make_async_copy(if_hbm.at[pl.ds(0, CHUNK)],
                                  irows.at[slot], isems[slot]).wait()

        fire_items(0, 0)
        for cp_ in bias_copies:
            cp_.wait()
        gb = gbv[...]

        for s in range(RING):
            fire_user(s, s)

        for j in range(nchunks):
            slot = j % 2
            drain_items(slot)
            if j + 1 < nchunks:
                fire_items(j + 1, 1 - slot)
            i_slot = irows.at[slot]

            @pl.loop(0, CHUNK, step=RING)
            def _(r):
                for s in range(RING):
                    rr = r + s
                    row = j * CHUNK + rr
                    drain_user(s)
                    rid = uid_at(row)
                    lane = jnp.full((LANES,), rid & 127, jnp.int32)
                    acc = jnp.zeros((LANES,), jnp.float32)
                    for t in range(D // LANES):
                        dsl = iota16 + t * LANES
                        pu = plsc.load_gather(ublk.at[s], [dsl, lane])
                        pv = i_slot[rr, pl.ds(t * LANES, LANES)]
                        acc = acc + pu * pv
                    dot = jnp.sum(acc)
                    row16 = jnp.full((LANES,), row, jnp.int32)
                    ub16 = plsc.load_gather(ubv, [row16])
                    ib16 = plsc.load_gather(ibv, [row16])
                    val = dot + ub16 + ib16 + gb
                    plsc.store_scatter(outv, [row16], val,
                                       mask=iota16 == 0)

                    nxt = row + RING
                    @pl.when(nxt < bpw)
                    def _():
                        nid = uid_at(nxt)
                        blk = pl.multiple_of((nid >> 7) << 7, CHUNK)
                        pltpu.async_copy(uf_hbm.at[:, pl.ds(blk, CHUNK)],
                                         ublk.at[s], usems[s])

        pltpu.sync_copy(outv, out_hbm.at[pl.ds(base, bpw)])

    return mf_kernel(uids, iids, uf_t, item_factors, ub_flat,
                     ib_flat, gb_vec)


# speedup vs baseline: 2.8544x; 1.3448x over previous
import dataclasses
import functools

import jax
import jax.numpy as jnp
from jax import lax
from jax.experimental import pallas as pl
from jax.experimental.pallas import tpu as pltpu
from jax.experimental.pallas import tpu_sc as plsc

NC = 2
NS = 16
NW = NC * NS
LANES = 16
CHUNK = 128
RING = 8


def kernel(user_ids, item_ids, user_factors, item_factors, user_biases,
           item_biases, global_bias):
    B = user_ids.shape[0]
    D = user_factors.shape[1]
    bpw = B // NW
    nchunks = bpw // CHUNK
    gpc = CHUNK // LANES

    uids = user_ids.astype(jnp.int32).reshape(B // CHUNK, CHUNK)
    iids = item_ids.astype(jnp.int32).reshape(B // CHUNK, CHUNK)
    uf_t = user_factors.T
    ub_t = user_biases.T
    ib_t = item_biases.T
    gb_vec = jnp.broadcast_to(global_bias.reshape(()), (LANES,))

    mesh = plsc.VectorSubcoreMesh(core_axis_name="c", subcore_axis_name="s")

    cp = pltpu.CompilerParams()
    for field, val in (("needs_layout_passes", False),
                       ("use_tc_tiling_on_sc", True)):
        if field in pltpu.CompilerParams.__dataclass_fields__:
            cp = dataclasses.replace(cp, **{field: val})

    @functools.partial(
        pl.kernel,
        out_type=jax.ShapeDtypeStruct((B,), jnp.float32),
        mesh=mesh,
        compiler_params=cp,
        scratch_types=[
            pltpu.VMEM((nchunks, CHUNK), jnp.int32),
            pltpu.VMEM((nchunks, CHUNK), jnp.int32),
            pltpu.VMEM((RING, D, CHUNK), jnp.float32),
            pltpu.VMEM((2, CHUNK, D), jnp.float32),
            pltpu.VMEM((bpw,), jnp.float32),
            pltpu.VMEM((bpw,), jnp.float32),
            pltpu.VMEM((bpw,), jnp.float32),
            pltpu.VMEM((LANES,), jnp.float32),
            pltpu.SemaphoreType.DMA,
            pltpu.SemaphoreType.DMA,
            pltpu.SemaphoreType.DMA,
        ] + [pltpu.SemaphoreType.DMA] * RING,
    )
    def mf_kernel(uid_hbm, iid_hbm, uf_hbm, if_hbm, ub_hbm, ib_hbm, gb_hbm,
                  out_hbm, uid_v, iid_v, ublk, irows, ubv, ibv, outv, gbv,
                  sem_b, sem_i0, sem_i1, *usems):
        wid = lax.axis_index("s") * NC + lax.axis_index("c")
        base = wid * bpw
        isems = (sem_i0, sem_i1)

        idx_rows = pl.ds(wid * nchunks, nchunks)
        pltpu.sync_copy(uid_hbm.at[idx_rows], uid_v)
        pltpu.sync_copy(iid_hbm.at[idx_rows], iid_v)
        pltpu.sync_copy(gb_hbm, gbv)

        bias_copies = []
        for j in range(nchunks):
            sl = pl.ds(j * CHUNK, CHUNK)
            bias_copies.append(
                pltpu.async_copy(ub_hbm.at[0].at[uid_v.at[j]],
                                 ubv.at[sl], sem_b))
            bias_copies.append(
                pltpu.async_copy(ib_hbm.at[0].at[iid_v.at[j]],
                                 ibv.at[sl], sem_b))

        iota16 = lax.iota(jnp.int32, LANES)
        zeros16 = jnp.zeros((LANES,), jnp.int32)

        def uid_at(r):
            grp = (r >> 4) << 4
            lane = r & 15
            vec = uid_v[r >> 7, pl.ds(grp & 127, LANES)]
            return jnp.sum(jnp.where(iota16 == lane, vec, 0))

        def fire_user(r, s):
            rid = uid_at(r)
            blk = pl.multiple_of((rid >> 7) << 7, CHUNK)
            pltpu.async_copy(uf_hbm.at[:, pl.ds(blk, CHUNK)],
                             ublk.at[s], usems[s])

        def drain_user(s):
            pltpu.make_async_copy(uf_hbm.at[:, pl.ds(0, CHUNK)],
                                  ublk.at[s], usems[s]).wait()

        def fire_items(j, slot):
            @pl.loop(0, gpc)
            def _(g):
                ivec = iid_v[j, pl.ds(g * LANES, LANES)]
                for lane in range(LANES):
                    rid = jnp.sum(jnp.where(iota16 == lane, ivec, 0))
                    dst = pl.ds(g * LANES + lane, 1)
                    pltpu.async_copy(if_hbm.at[pl.ds(rid, 1)],
                                     irows.at[slot].at[dst], isems[slot])

        def drain_items(slot):
            pltpu.make_async_copy(if_hbm.at[pl.ds(0, CHUNK)],
                                  irows.at[slot], isems[slot]).wait()

        fire_items(0, 0)
        for cp_ in bias_copies:
            cp_.wait()
        gb = gbv[...]

        for s in range(RING):
            fire_user(s, s)

        for j in range(nchunks):
            slot = j % 2
            drain_items(slot)
            if j + 1 < nchunks:
                fire_items(j + 1, 1 - slot)
            i_slot = irows.at[slot]

            @pl.loop(0, CHUNK, step=RING)
            def _(r):
                for s in range(RING):
                    rr = r + s
                    row = j * CHUNK + rr
                    drain_user(s)
                    rid = uid_at(row)
                    lane = jnp.full((LANES,), rid & 127, jnp.int32)
                    acc = jnp.zeros((LANES,), jnp.float32)
                    for t in range(D // LANES):
                        dsl = iota16 + t * LANES
                        pu = plsc.load_gather(ublk.at[s], [dsl, lane])
                        pv = i_slot[rr, pl.ds(t * LANES, LANES)]
                        acc = acc + pu * pv
                    dot = jnp.sum(acc)
                    row16 = jnp.full((LANES,), row, jnp.int32)
                    ub16 = plsc.load_gather(ubv, [row16])
                    ib16 = plsc.load_gather(ibv, [row16])
                    val = dot + ub16 + ib16 + gb
                    plsc.store_scatter(outv, [row16], val,
                                       mask=iota16 == 0)

                    nxt = row + RING
                    @pl.when(nxt < bpw)
                    def _():
                        nid = uid_at(nxt)
                        blk = pl.multiple_of((nid >> 7) << 7, CHUNK)
                        pltpu.async_copy(uf_hbm.at[:, pl.ds(blk, CHUNK)],
                                         ublk.at[s], usems[s])

        pltpu.sync_copy(outv, out_hbm.at[pl.ds(base, bpw)])

    return mf_kernel(uids, iids, uf_t, item_factors, ub_t,
                     ib_t, gb_vec)
